# baseline (speedup 1.0000x reference)
import functools

import jax
import jax.numpy as jnp
from jax import lax
from jax.experimental import pallas as pl
from jax.experimental.pallas import tpu as pltpu
from jax.experimental.pallas import tpu_sc as plsc

N = 10000
E = 320000
NG = 64
DIM = 64
BN_EPS = 1e-5

NC = 2
NS = 16
NW = NC * NS
EDGES_PER_W = E // NW
NBLK = 5
NBUF = 5
DRAIN = 2
NP = 10240
ROWS_PER_S = NP // NS


def _make_seg_sum(D, chunk):
  nchunks = EDGES_PER_W // chunk
  iblk = nchunks // NBLK
  assert iblk % NBUF == 0 and chunk % 8 == 0 and chunk <= 128
  mesh = plsc.VectorSubcoreMesh(core_axis_name="c", subcore_axis_name="s")
  scratch = ([
      pltpu.VMEM((iblk, chunk), jnp.int32),
      pltpu.VMEM((iblk, chunk), jnp.int32),
      pltpu.VMEM((iblk, chunk), jnp.int32),
      pltpu.VMEM((iblk, chunk), jnp.int32),
      pltpu.SemaphoreType.DMA,
      pltpu.SemaphoreType.DMA,
  ] + [pltpu.VMEM((chunk, D), jnp.float32) for _ in range(NBUF)]
    + [pltpu.SemaphoreType.DMA for _ in range(2 * NBUF)]
    + [pltpu.VMEM_SHARED((NP, D), jnp.float32)])

  @functools.partial(
      pl.kernel,
      mesh=mesh,
      out_type=jax.ShapeDtypeStruct((NC * NP, D), jnp.float32),
      scratch_types=scratch,
      compiler_params=pltpu.CompilerParams(use_tc_tiling_on_sc=False),
      name=f"gin_seg_sum_d{D}",
  )
  def seg_sum(x_hbm, src_hbm, dst_hbm, zeros_hbm, out_hbm,
              is0, id0, is1, id1, semi0, semi1, *rest):
    rows = rest[:NBUF]
    sems = rest[NBUF:2 * NBUF]
    sems_s = rest[2 * NBUF:3 * NBUF]
    acc = rest[3 * NBUF]
    idx = [(is0, id0, semi0), (is1, id1, semi1)]
    c = lax.axis_index("c")
    s = lax.axis_index("s")
    wid = c * NS + s

    pltpu.sync_copy(src_hbm.at[wid, 0], is0)
    pltpu.sync_copy(dst_hbm.at[wid, 0], id0)

    r0 = pl.multiple_of(s * ROWS_PER_S, 8)
    pltpu.sync_copy(zeros_hbm.at[pl.ds(r0, ROWS_PER_S)],
                    acc.at[pl.ds(r0, ROWS_PER_S)])
    plsc.subcore_barrier()

    for blk in range(NBLK):
      isv, idv, _ = idx[blk % 2]
      if blk + 1 < NBLK:
        nsv, ndv, nsem = idx[(blk + 1) % 2]
        pltpu.async_copy(src_hbm.at[wid, blk + 1], nsv, nsem)
        pltpu.async_copy(dst_hbm.at[wid, blk + 1], ndv, nsem)

      for b in range(NBUF - DRAIN):
        pltpu.async_copy(x_hbm.at[isv.at[b]], rows[b], sems[b])

      def inner(it, carry):
        jbase = it * NBUF
        for b in range(NBUF):
          jj = jbase + b
          bd = (b - DRAIN) % NBUF

          @pl.when(jj >= DRAIN)
          def _():
            pltpu.make_async_copy(rows[bd], acc.at[idv.at[jj]],
                                  sems_s[bd]).wait()

          @pl.when(jj + NBUF - DRAIN < iblk)
          def _():
            pltpu.async_copy(x_hbm.at[isv.at[jj + NBUF - DRAIN]], rows[bd],
                             sems[bd])

          pltpu.make_async_copy(x_hbm.at[isv.at[jj]], rows[b],
                                sems[b]).wait()
          pltpu.async_copy(rows[b], acc.at[idv.at[jj]], sems_s[b], add=True)
        return carry

      lax.fori_loop(0, iblk // NBUF, inner, 0, unroll=False)
      for k in range(DRAIN):
        bt = (iblk - DRAIN + k) % NBUF
        pltpu.make_async_copy(rows[bt], acc.at[idv.at[0]], sems_s[bt]).wait()

      if blk + 1 < NBLK:
        nsv, ndv, nsem = idx[(blk + 1) % 2]
        pltpu.make_async_copy(src_hbm.at[wid, blk + 1], nsv, nsem).wait()
        pltpu.make_async_copy(dst_hbm.at[wid, blk + 1], ndv, nsem).wait()

    plsc.subcore_barrier()
    out0 = pl.multiple_of(c * NP + s * ROWS_PER_S, 8)
    pltpu.sync_copy(acc.at[pl.ds(r0, ROWS_PER_S)],
                    out_hbm.at[pl.ds(out0, ROWS_PER_S)])

  return seg_sum


_seg_sum_cache = {}


def _seg_sum(D):
  if D not in _seg_sum_cache:
    _seg_sum_cache[D] = _make_seg_sum(D, 40 if D > 64 else 80)
  return _seg_sum_cache[D]


def _bn_mlp_pool(x_ref, agg_ref, gi_ref, w1_ref, b1_ref, w2_ref, b2_ref,
                 gam_ref, bet_ref):
  h = x_ref[...] + (agg_ref[pl.ds(0, N), :] + agg_ref[pl.ds(NP, N), :])
  h = jnp.maximum(
      jnp.dot(h, w1_ref[...],
              preferred_element_type=jnp.float32) + b1_ref[...], 0.0)
  h = jnp.dot(h, w2_ref[...],
              preferred_element_type=jnp.float32) + b2_ref[...]
  h = jnp.maximum(h, 0.0)
  mean = jnp.mean(h, axis=0, keepdims=True)
  var = jnp.mean((h - mean) ** 2, axis=0, keepdims=True)
  v = var + BN_EPS
  inv = lax.rsqrt(v)
  inv = inv * (1.5 - 0.5 * v * inv * inv)
  inv = inv * (1.5 - 0.5 * v * inv * inv)
  y = (h - mean) * (inv * gam_ref[...]) + bet_ref[...]
  gids = lax.broadcasted_iota(jnp.int32, (NG, 1), 0)
  mask = (gi_ref[...] == gids).astype(jnp.float32)
  pool = jnp.dot(mask, y, preferred_element_type=jnp.float32)
  return y, pool


def _tc_layer_body(x_ref, agg_ref, gi_ref, w1_ref, b1_ref, w2_ref, b2_ref,
                   gam_ref, bet_ref, y_ref, pool_ref):
  y, pool = _bn_mlp_pool(x_ref, agg_ref, gi_ref, w1_ref, b1_ref, w2_ref,
                         b2_ref, gam_ref, bet_ref)
  y_ref[...] = y
  pool_ref[...] = pool


def _tc_last_body(x_ref, agg_ref, gi_ref, w1_ref, b1_ref, w2_ref, b2_ref,
                  gam_ref, bet_ref, y0, y1, y2, y3, p0, p1, p2, p3,
                  xs_ref, xo_ref):
  y, pool = _bn_mlp_pool(x_ref, agg_ref, gi_ref, w1_ref, b1_ref, w2_ref,
                         b2_ref, gam_ref, bet_ref)
  xs_ref[...] = jnp.concatenate(
      [y0[...], y1[...], y2[...], y3[...], y], axis=1)
  xo_ref[...] = jnp.concatenate(
      [p0[...], p1[...], p2[...], p3[...], pool], axis=1)


def _tc_layer(x, agg2, gi2d, p):
  return pl.pallas_call(
      _tc_layer_body,
      out_shape=[
          jax.ShapeDtypeStruct((N, DIM), jnp.float32),
          jax.ShapeDtypeStruct((NG, DIM), jnp.float32),
      ],
  )(x, agg2, gi2d, p["W1"], p["b1"].reshape(1, DIM), p["W2"],
    p["b2"].reshape(1, DIM), p["gamma"].reshape(1, DIM),
    p["beta"].reshape(1, DIM))


def _tc_last(x, agg2, gi2d, p, ys, pools):
  return pl.pallas_call(
      _tc_last_body,
      out_shape=[
          jax.ShapeDtypeStruct((N, 5 * DIM), jnp.float32),
          jax.ShapeDtypeStruct((NG, 5 * DIM), jnp.float32),
      ],
  )(x, agg2, gi2d, p["W1"], p["b1"].reshape(1, DIM), p["W2"],
    p["b2"].reshape(1, DIM), p["gamma"].reshape(1, DIM),
    p["beta"].reshape(1, DIM), *ys, *pools)


def kernel(node_features, edge_index, graph_index, params):
  gi2d = graph_index.reshape(1, N)
  x = node_features
  xs = []
  pools = []
  d0 = node_features.shape[1]
  zeros = {d: jnp.zeros((NP, d), jnp.float32) for d in (d0, DIM)}
  edge4 = {}
  for d in (d0, DIM):
    chunk = 40 if d > 64 else 80
    nchunks = EDGES_PER_W // chunk
    edge4[d] = (edge_index[0].reshape(NW, NBLK, nchunks // NBLK, chunk),
                edge_index[1].reshape(NW, NBLK, nchunks // NBLK, chunk))
  for i in range(4):
    p = params[f"layer_{i}"]
    d = x.shape[1]
    src4, dst4 = edge4[d]
    agg2 = _seg_sum(d)(x, src4, dst4, zeros[d])
    y, pooled = _tc_layer(x, agg2, gi2d, p)
    xs.append(y)
    pools.append(pooled)
    x = y
  src4, dst4 = edge4[DIM]
  agg2 = _seg_sum(DIM)(x, src4, dst4, zeros[DIM])
  xs_out, x_out = _tc_last(x, agg2, gi2d, params["layer_4"], xs, pools)
  return x_out, xs_out


# speedup vs baseline: 1.0106x; 1.0106x over previous
import functools

import jax
import jax.numpy as jnp
from jax import lax
from jax.experimental import pallas as pl
from jax.experimental.pallas import tpu as pltpu
from jax.experimental.pallas import tpu_sc as plsc

N = 10000
E = 320000
NG = 64
DIM = 64
BN_EPS = 1e-5

NC = 2
NS = 16
NW = NC * NS
EDGES_PER_W = E // NW
NBLK = 5
NBUF = 5
DRAIN = 2
NP = 10240
ROWS_PER_S = NP // NS


def _make_seg_sum(D, chunk):
  nchunks = EDGES_PER_W // chunk
  iblk = nchunks // NBLK
  assert iblk % NBUF == 0 and chunk % 8 == 0 and chunk <= 128
  mesh = plsc.VectorSubcoreMesh(core_axis_name="c", subcore_axis_name="s")
  scratch = ([
      pltpu.VMEM((iblk, chunk), jnp.int32),
      pltpu.VMEM((iblk, chunk), jnp.int32),
      pltpu.VMEM((iblk, chunk), jnp.int32),
      pltpu.VMEM((iblk, chunk), jnp.int32),
      pltpu.SemaphoreType.DMA,
      pltpu.SemaphoreType.DMA,
  ] + [pltpu.VMEM((chunk, D), jnp.float32) for _ in range(NBUF)]
    + [pltpu.SemaphoreType.DMA for _ in range(2 * NBUF)]
    + [pltpu.VMEM_SHARED((NP, D), jnp.float32)])

  @functools.partial(
      pl.kernel,
      mesh=mesh,
      out_type=jax.ShapeDtypeStruct((NC * NP, D), jnp.float32),
      scratch_types=scratch,
      compiler_params=pltpu.CompilerParams(use_tc_tiling_on_sc=False),
      name=f"gin_seg_sum_d{D}",
  )
  def seg_sum(x_hbm, src_hbm, dst_hbm, zeros_hbm, out_hbm,
              is0, id0, is1, id1, semi0, semi1, *rest):
    rows = rest[:NBUF]
    sems = rest[NBUF:2 * NBUF]
    sems_s = rest[2 * NBUF:3 * NBUF]
    acc = rest[3 * NBUF]
    idx = [(is0, id0, semi0), (is1, id1, semi1)]
    c = lax.axis_index("c")
    s = lax.axis_index("s")
    wid = c * NS + s

    pltpu.sync_copy(src_hbm.at[wid, 0], is0)
    pltpu.sync_copy(dst_hbm.at[wid, 0], id0)

    r0 = pl.multiple_of(s * ROWS_PER_S, 8)
    pltpu.sync_copy(zeros_hbm.at[pl.ds(r0, ROWS_PER_S)],
                    acc.at[pl.ds(r0, ROWS_PER_S)])
    plsc.subcore_barrier()

    for blk in range(NBLK):
      isv, idv, _ = idx[blk % 2]
      if blk + 1 < NBLK:
        nsv, ndv, nsem = idx[(blk + 1) % 2]
        pltpu.async_copy(src_hbm.at[wid, blk + 1], nsv, nsem)
        pltpu.async_copy(dst_hbm.at[wid, blk + 1], ndv, nsem)

      for b in range(NBUF):
        pltpu.async_copy(x_hbm.at[isv.at[b]], rows[b], sems[b])

      def inner(it, carry):
        jbase = it * NBUF
        for b in range(NBUF):
          jj = jbase + b
          pltpu.make_async_copy(x_hbm.at[isv.at[jj]], rows[b],
                                sems[b]).wait()
          pltpu.sync_copy(rows[b], acc.at[idv.at[jj]], add=True)

          @pl.when(jj + NBUF < iblk)
          def _():
            pltpu.async_copy(x_hbm.at[isv.at[jj + NBUF]], rows[b], sems[b])
        return carry

      lax.fori_loop(0, iblk // NBUF, inner, 0, unroll=False)

      if blk + 1 < NBLK:
        nsv, ndv, nsem = idx[(blk + 1) % 2]
        pltpu.make_async_copy(src_hbm.at[wid, blk + 1], nsv, nsem).wait()
        pltpu.make_async_copy(dst_hbm.at[wid, blk + 1], ndv, nsem).wait()

    plsc.subcore_barrier()
    out0 = pl.multiple_of(c * NP + s * ROWS_PER_S, 8)
    pltpu.sync_copy(acc.at[pl.ds(r0, ROWS_PER_S)],
                    out_hbm.at[pl.ds(out0, ROWS_PER_S)])

  return seg_sum


_seg_sum_cache = {}


def _seg_sum(D):
  if D not in _seg_sum_cache:
    _seg_sum_cache[D] = _make_seg_sum(D, 40 if D > 64 else 80)
  return _seg_sum_cache[D]


def _bn_mlp_pool(x_ref, agg_ref, gi_ref, w1_ref, b1_ref, w2_ref, b2_ref,
                 gam_ref, bet_ref):
  h = x_ref[...] + (agg_ref[pl.ds(0, N), :] + agg_ref[pl.ds(NP, N), :])
  h = jnp.maximum(
      jnp.dot(h, w1_ref[...],
              preferred_element_type=jnp.float32) + b1_ref[...], 0.0)
  h = jnp.dot(h, w2_ref[...],
              preferred_element_type=jnp.float32) + b2_ref[...]
  h = jnp.maximum(h, 0.0)
  mean = jnp.mean(h, axis=0, keepdims=True)
  var = jnp.mean((h - mean) ** 2, axis=0, keepdims=True)
  v = var + BN_EPS
  inv = lax.rsqrt(v)
  inv = inv * (1.5 - 0.5 * v * inv * inv)
  inv = inv * (1.5 - 0.5 * v * inv * inv)
  y = (h - mean) * (inv * gam_ref[...]) + bet_ref[...]
  gids = lax.broadcasted_iota(jnp.int32, (NG, 1), 0)
  mask = (gi_ref[...] == gids).astype(jnp.float32)
  pool = jnp.dot(mask, y, preferred_element_type=jnp.float32)
  return y, pool


def _tc_layer_body(x_ref, agg_ref, gi_ref, w1_ref, b1_ref, w2_ref, b2_ref,
                   gam_ref, bet_ref, y_ref, pool_ref):
  y, pool = _bn_mlp_pool(x_ref, agg_ref, gi_ref, w1_ref, b1_ref, w2_ref,
                         b2_ref, gam_ref, bet_ref)
  y_ref[...] = y
  pool_ref[...] = pool


def _tc_last_body(x_ref, agg_ref, gi_ref, w1_ref, b1_ref, w2_ref, b2_ref,
                  gam_ref, bet_ref, y0, y1, y2, y3, p0, p1, p2, p3,
                  xs_ref, xo_ref):
  y, pool = _bn_mlp_pool(x_ref, agg_ref, gi_ref, w1_ref, b1_ref, w2_ref,
                         b2_ref, gam_ref, bet_ref)
  xs_ref[...] = jnp.concatenate(
      [y0[...], y1[...], y2[...], y3[...], y], axis=1)
  xo_ref[...] = jnp.concatenate(
      [p0[...], p1[...], p2[...], p3[...], pool], axis=1)


def _tc_layer(x, agg2, gi2d, p):
  return pl.pallas_call(
      _tc_layer_body,
      out_shape=[
          jax.ShapeDtypeStruct((N, DIM), jnp.float32),
          jax.ShapeDtypeStruct((NG, DIM), jnp.float32),
      ],
  )(x, agg2, gi2d, p["W1"], p["b1"].reshape(1, DIM), p["W2"],
    p["b2"].reshape(1, DIM), p["gamma"].reshape(1, DIM),
    p["beta"].reshape(1, DIM))


def _tc_last(x, agg2, gi2d, p, ys, pools):
  return pl.pallas_call(
      _tc_last_body,
      out_shape=[
          jax.ShapeDtypeStruct((N, 5 * DIM), jnp.float32),
          jax.ShapeDtypeStruct((NG, 5 * DIM), jnp.float32),
      ],
  )(x, agg2, gi2d, p["W1"], p["b1"].reshape(1, DIM), p["W2"],
    p["b2"].reshape(1, DIM), p["gamma"].reshape(1, DIM),
    p["beta"].reshape(1, DIM), *ys, *pools)


def kernel(node_features, edge_index, graph_index, params):
  gi2d = graph_index.reshape(1, N)
  x = node_features
  xs = []
  pools = []
  d0 = node_features.shape[1]
  zeros = {d: jnp.zeros((NP, d), jnp.float32) for d in (d0, DIM)}
  edge4 = {}
  for d in (d0, DIM):
    chunk = 40 if d > 64 else 80
    nchunks = EDGES_PER_W // chunk
    edge4[d] = (edge_index[0].reshape(NW, NBLK, nchunks // NBLK, chunk),
                edge_index[1].reshape(NW, NBLK, nchunks // NBLK, chunk))
  for i in range(4):
    p = params[f"layer_{i}"]
    d = x.shape[1]
    src4, dst4 = edge4[d]
    agg2 = _seg_sum(d)(x, src4, dst4, zeros[d])
    y, pooled = _tc_layer(x, agg2, gi2d, p)
    xs.append(y)
    pools.append(pooled)
    x = y
  src4, dst4 = edge4[DIM]
  agg2 = _seg_sum(DIM)(x, src4, dst4, zeros[DIM])
  xs_out, x_out = _tc_last(x, agg2, gi2d, params["layer_4"], xs, pools)
  return x_out, xs_out


# speedup vs baseline: 1.1011x; 1.0895x over previous
import functools

import jax
import jax.numpy as jnp
from jax import lax
from jax.experimental import pallas as pl
from jax.experimental.pallas import tpu as pltpu
from jax.experimental.pallas import tpu_sc as plsc

N = 10000
E = 320000
NG = 64
DIM = 64
BN_EPS = 1e-5

NC = 2
NS = 16
NW = NC * NS
EDGES_PER_W = E // NW
NBLK = 5
NBUF = 5
DRAIN = 2
NP = 10240
ROWS_PER_S = NP // NS


def _make_seg_sum(D, chunk):
  nchunks = EDGES_PER_W // chunk
  iblk = nchunks // NBLK
  assert iblk % NBUF == 0 and chunk % 8 == 0 and chunk <= 128
  mesh = plsc.VectorSubcoreMesh(core_axis_name="c", subcore_axis_name="s")
  scratch = ([
      pltpu.VMEM((iblk, chunk), jnp.int32),
      pltpu.VMEM((iblk, chunk), jnp.int32),
      pltpu.VMEM((iblk, chunk), jnp.int32),
      pltpu.VMEM((iblk, chunk), jnp.int32),
      pltpu.SemaphoreType.DMA,
      pltpu.SemaphoreType.DMA,
  ] + [pltpu.VMEM((chunk, D), jnp.float32) for _ in range(NBUF)]
    + [pltpu.SemaphoreType.DMA for _ in range(2 * NBUF)]
    + [pltpu.VMEM_SHARED((NP, D), jnp.float32)])

  @functools.partial(
      pl.kernel,
      mesh=mesh,
      out_type=jax.ShapeDtypeStruct((NC * NP, D), jnp.float32),
      scratch_types=scratch,
      compiler_params=pltpu.CompilerParams(use_tc_tiling_on_sc=False),
      name=f"gin_seg_sum_d{D}",
  )
  def seg_sum(x_hbm, src_hbm, dst_hbm, zeros_hbm, out_hbm,
              is0, id0, is1, id1, semi0, semi1, *rest):
    rows = rest[:NBUF]
    sems = rest[NBUF:2 * NBUF]
    sems_s = rest[2 * NBUF:3 * NBUF]
    acc = rest[3 * NBUF]
    idx = [(is0, id0, semi0), (is1, id1, semi1)]
    c = lax.axis_index("c")
    s = lax.axis_index("s")
    wid = c * NS + s

    pltpu.sync_copy(src_hbm.at[wid, 0], is0)
    pltpu.sync_copy(dst_hbm.at[wid, 0], id0)

    r0 = pl.multiple_of(s * ROWS_PER_S, 8)
    pltpu.sync_copy(zeros_hbm.at[pl.ds(r0, ROWS_PER_S)],
                    acc.at[pl.ds(r0, ROWS_PER_S)])
    plsc.subcore_barrier()

    for blk in range(NBLK):
      isv, idv, _ = idx[blk % 2]
      if blk + 1 < NBLK:
        nsv, ndv, nsem = idx[(blk + 1) % 2]
        pltpu.async_copy(src_hbm.at[wid, blk + 1], nsv, nsem)
        pltpu.async_copy(dst_hbm.at[wid, blk + 1], ndv, nsem)

      for b in range(NBUF):
        pltpu.async_copy(x_hbm.at[isv.at[b]], rows[b], sems[b])

      def inner(it, carry):
        jbase = it * NBUF
        for b in range(NBUF):
          jj = jbase + b
          pltpu.make_async_copy(x_hbm.at[isv.at[jj]], rows[b],
                                sems[b]).wait()
          pltpu.sync_copy(rows[b], acc.at[idv.at[jj]], add=True)

          @pl.when(jj + NBUF < iblk)
          def _():
            pltpu.async_copy(x_hbm.at[isv.at[jj + NBUF]], rows[b], sems[b])
        return carry

      lax.fori_loop(0, iblk // NBUF, inner, 0, unroll=False)

      if blk + 1 < NBLK:
        nsv, ndv, nsem = idx[(blk + 1) % 2]
        pltpu.make_async_copy(src_hbm.at[wid, blk + 1], nsv, nsem).wait()
        pltpu.make_async_copy(dst_hbm.at[wid, blk + 1], ndv, nsem).wait()

    plsc.subcore_barrier()
    out0 = pl.multiple_of(c * NP + s * ROWS_PER_S, 8)
    pltpu.sync_copy(acc.at[pl.ds(r0, ROWS_PER_S)],
                    out_hbm.at[pl.ds(out0, ROWS_PER_S)])

  return seg_sum


_seg_sum_cache = {}


def _seg_sum(D):
  if D not in _seg_sum_cache:
    _seg_sum_cache[D] = _make_seg_sum(D, 40 if D > 64 else 80)
  return _seg_sum_cache[D]


NH = N // 2
NPH = NP // 2


def _bd(w_ref):
  w = w_ref[...]
  z = jnp.zeros_like(w)
  return jnp.concatenate(
      [jnp.concatenate([w, z], axis=1), jnp.concatenate([z, w], axis=1)],
      axis=0)


def _t2(v):
  return jnp.concatenate([v, v], axis=1)


def _pack128(y):
  v = jnp.reshape(y, (NH, 2, DIM))
  return jnp.concatenate([v[:, 0, :], v[:, 1, :]], axis=1)


def _unpack128(y128):
  s = jnp.stack([y128[:, :DIM], y128[:, DIM:]], axis=1)
  return jnp.reshape(s, (N, DIM))


def _mlp_bn128(x128, agg_ref, w1_ref, b1_ref, w2_ref, b2_ref,
               gam_ref, bet_ref):
  h = x128 + (agg_ref[pl.ds(0, NH), :] + agg_ref[pl.ds(NPH, NH), :])
  h = jnp.maximum(
      jnp.dot(h, _bd(w1_ref),
              preferred_element_type=jnp.float32) + _t2(b1_ref[...]), 0.0)
  h = jnp.dot(h, _bd(w2_ref),
              preferred_element_type=jnp.float32) + _t2(b2_ref[...])
  h = jnp.maximum(h, 0.0)
  m128 = jnp.mean(h, axis=0, keepdims=True)
  m = _t2(0.5 * (m128[:, :DIM] + m128[:, DIM:]))
  hc = h - m
  v128 = jnp.mean(hc * hc, axis=0, keepdims=True)
  v = 0.5 * (v128[:, :DIM] + v128[:, DIM:]) + BN_EPS
  inv = lax.rsqrt(v)
  inv = inv * (1.5 - 0.5 * v * inv * inv)
  inv = inv * (1.5 - 0.5 * v * inv * inv)
  y = hc * _t2(inv * gam_ref[...]) + _t2(bet_ref[...])
  return y


def _pool128(y, ge_ref, go_ref):
  gids = lax.broadcasted_iota(jnp.int32, (NG, 1), 0)
  me = (ge_ref[...] == gids).astype(jnp.float32)
  mo = (go_ref[...] == gids).astype(jnp.float32)
  pe = jnp.dot(me, y, preferred_element_type=jnp.float32)
  po = jnp.dot(mo, y, preferred_element_type=jnp.float32)
  return pe[:, :DIM] + po[:, DIM:]


def _bn_mlp_pool(x_ref, agg_ref, gi_ref, w1_ref, b1_ref, w2_ref, b2_ref,
                 gam_ref, bet_ref):
  h = x_ref[...] + (agg_ref[pl.ds(0, N), :] + agg_ref[pl.ds(NP, N), :])
  h = jnp.maximum(
      jnp.dot(h, w1_ref[...],
              preferred_element_type=jnp.float32) + b1_ref[...], 0.0)
  h = jnp.dot(h, w2_ref[...],
              preferred_element_type=jnp.float32) + b2_ref[...]
  h = jnp.maximum(h, 0.0)
  mean = jnp.mean(h, axis=0, keepdims=True)
  var = jnp.mean((h - mean) ** 2, axis=0, keepdims=True)
  v = var + BN_EPS
  inv = lax.rsqrt(v)
  inv = inv * (1.5 - 0.5 * v * inv * inv)
  inv = inv * (1.5 - 0.5 * v * inv * inv)
  y = (h - mean) * (inv * gam_ref[...]) + bet_ref[...]
  gids = lax.broadcasted_iota(jnp.int32, (NG, 1), 0)
  mask = (gi_ref[...] == gids).astype(jnp.float32)
  pool = jnp.dot(mask, y, preferred_element_type=jnp.float32)
  return y, pool


def _tc_first_body(x_ref, agg_ref, gi_ref, w1_ref, b1_ref, w2_ref, b2_ref,
                   gam_ref, bet_ref, y_ref, y128_ref, pool_ref):
  y, pool = _bn_mlp_pool(x_ref, agg_ref, gi_ref, w1_ref, b1_ref, w2_ref,
                         b2_ref, gam_ref, bet_ref)
  y_ref[...] = y
  y128_ref[...] = _pack128(y)
  pool_ref[...] = pool


def _tc_mid_body(x_ref, agg_ref, ge_ref, go_ref, w1_ref, b1_ref, w2_ref,
                 b2_ref, gam_ref, bet_ref, y_ref, y128_ref, pool_ref):
  y = _mlp_bn128(x_ref[...], agg_ref, w1_ref, b1_ref, w2_ref, b2_ref,
                 gam_ref, bet_ref)
  y128_ref[...] = y
  y_ref[...] = _unpack128(y)
  pool_ref[...] = _pool128(y, ge_ref, go_ref)


def _tc_last_body(x_ref, agg_ref, ge_ref, go_ref, w1_ref, b1_ref, w2_ref,
                  b2_ref, gam_ref, bet_ref, y0, y1, y2, y3, p0, p1, p2, p3,
                  xs_ref, xo_ref):
  y = _mlp_bn128(x_ref[...], agg_ref, w1_ref, b1_ref, w2_ref, b2_ref,
                 gam_ref, bet_ref)
  pool = _pool128(y, ge_ref, go_ref)
  xs_ref[...] = jnp.concatenate(
      [y0[...], y1[...], y2[...], y3[...], _unpack128(y)], axis=1)
  xo_ref[...] = jnp.concatenate(
      [p0[...], p1[...], p2[...], p3[...], pool], axis=1)


def _wparams(p):
  return (p["W1"], p["b1"].reshape(1, DIM), p["W2"], p["b2"].reshape(1, DIM),
          p["gamma"].reshape(1, DIM), p["beta"].reshape(1, DIM))


def _tc_first(x, agg2, gi2d, p):
  return pl.pallas_call(
      _tc_first_body,
      out_shape=[
          jax.ShapeDtypeStruct((N, DIM), jnp.float32),
          jax.ShapeDtypeStruct((NH, 2 * DIM), jnp.float32),
          jax.ShapeDtypeStruct((NG, DIM), jnp.float32),
      ],
  )(x, agg2, gi2d, *_wparams(p))


def _tc_mid(x128, agg128, ge, go, p):
  return pl.pallas_call(
      _tc_mid_body,
      out_shape=[
          jax.ShapeDtypeStruct((N, DIM), jnp.float32),
          jax.ShapeDtypeStruct((NH, 2 * DIM), jnp.float32),
          jax.ShapeDtypeStruct((NG, DIM), jnp.float32),
      ],
  )(x128, agg128, ge, go, *_wparams(p))


def _tc_last(x128, agg128, ge, go, p, ys, pools):
  return pl.pallas_call(
      _tc_last_body,
      out_shape=[
          jax.ShapeDtypeStruct((N, 5 * DIM), jnp.float32),
          jax.ShapeDtypeStruct((NG, 5 * DIM), jnp.float32),
      ],
  )(x128, agg128, ge, go, *_wparams(p), *ys, *pools)


def kernel(node_features, edge_index, graph_index, params):
  gi2d = graph_index.reshape(1, N)
  ge = graph_index[0::2].reshape(1, NH)
  go = graph_index[1::2].reshape(1, NH)
  d0 = node_features.shape[1]
  zeros = {d: jnp.zeros((NP, d), jnp.float32) for d in (d0, DIM)}
  edge4 = {}
  for d in (d0, DIM):
    chunk = 40 if d > 64 else 80
    nchunks = EDGES_PER_W // chunk
    edge4[d] = (edge_index[0].reshape(NW, NBLK, nchunks // NBLK, chunk),
                edge_index[1].reshape(NW, NBLK, nchunks // NBLK, chunk))

  src4, dst4 = edge4[d0]
  agg2 = _seg_sum(d0)(node_features, src4, dst4, zeros[d0])
  y0, x128, pool0 = _tc_first(node_features, agg2, gi2d, params["layer_0"])
  ys = [y0]
  pools = [pool0]
  src4, dst4 = edge4[DIM]
  for i in (1, 2, 3):
    agg2 = _seg_sum(DIM)(x128.reshape(N, DIM), src4, dst4, zeros[DIM])
    agg128 = agg2.reshape(NC * NPH, 2 * DIM)
    y64, x128, pooled = _tc_mid(x128, agg128, ge, go, params[f"layer_{i}"])
    ys.append(y64)
    pools.append(pooled)
  agg2 = _seg_sum(DIM)(x128.reshape(N, DIM), src4, dst4, zeros[DIM])
  agg128 = agg2.reshape(NC * NPH, 2 * DIM)
  xs_out, x_out = _tc_last(x128, agg128, ge, go, params["layer_4"], ys, pools)
  return x_out, xs_out


# speedup vs baseline: 1.1228x; 1.0198x over previous
import functools

import jax
import jax.numpy as jnp
from jax import lax
from jax.experimental import pallas as pl
from jax.experimental.pallas import tpu as pltpu
from jax.experimental.pallas import tpu_sc as plsc

N = 10000
E = 320000
NG = 64
DIM = 64
BN_EPS = 1e-5

NC = 2
NS = 16
NW = NC * NS
EDGES_PER_W = E // NW
NBLK = 5
NBUF = 5
DRAIN = 2
NP = 10240
ROWS_PER_S = NP // NS


def _make_seg_sum(D, chunk):
  nchunks = EDGES_PER_W // chunk
  iblk = nchunks // NBLK
  assert iblk % NBUF == 0 and chunk % 8 == 0 and chunk <= 128
  mesh = plsc.VectorSubcoreMesh(core_axis_name="c", subcore_axis_name="s")
  scratch = ([
      pltpu.VMEM((iblk, chunk), jnp.int32),
      pltpu.VMEM((iblk, chunk), jnp.int32),
      pltpu.VMEM((iblk, chunk), jnp.int32),
      pltpu.VMEM((iblk, chunk), jnp.int32),
      pltpu.SemaphoreType.DMA,
      pltpu.SemaphoreType.DMA,
  ] + [pltpu.VMEM((chunk, D), jnp.float32) for _ in range(NBUF)]
    + [pltpu.SemaphoreType.DMA for _ in range(2 * NBUF)]
    + [pltpu.VMEM_SHARED((NP, D), jnp.float32)])

  @functools.partial(
      pl.kernel,
      mesh=mesh,
      out_type=jax.ShapeDtypeStruct((NC * NP, D), jnp.float32),
      scratch_types=scratch,
      compiler_params=pltpu.CompilerParams(use_tc_tiling_on_sc=False),
      name=f"gin_seg_sum_d{D}",
  )
  def seg_sum(x_hbm, src_hbm, dst_hbm, zeros_hbm, out_hbm,
              is0, id0, is1, id1, semi0, semi1, *rest):
    rows = rest[:NBUF]
    sems = rest[NBUF:2 * NBUF]
    sems_s = rest[2 * NBUF:3 * NBUF]
    acc = rest[3 * NBUF]
    idx = [(is0, id0, semi0), (is1, id1, semi1)]
    c = lax.axis_index("c")
    s = lax.axis_index("s")
    wid = c * NS + s

    pltpu.async_copy(src_hbm.at[wid, 0], is0, semi0)
    pltpu.async_copy(dst_hbm.at[wid, 0], id0, semi0)
    r0 = pl.multiple_of(s * ROWS_PER_S, 8)
    pltpu.sync_copy(zeros_hbm.at[pl.ds(r0, ROWS_PER_S)],
                    acc.at[pl.ds(r0, ROWS_PER_S)])
    pltpu.make_async_copy(src_hbm.at[wid, 0], is0, semi0).wait()
    pltpu.make_async_copy(dst_hbm.at[wid, 0], id0, semi0).wait()
    for b in range(NBUF):
      pltpu.async_copy(x_hbm.at[is0.at[b]], rows[b], sems[b])
    plsc.subcore_barrier()

    for blk in range(NBLK):
      isv, idv, _ = idx[blk % 2]
      if blk + 1 < NBLK:
        nsv, ndv, nsem = idx[(blk + 1) % 2]
        pltpu.async_copy(src_hbm.at[wid, blk + 1], nsv, nsem)
        pltpu.async_copy(dst_hbm.at[wid, blk + 1], ndv, nsem)

      if blk > 0:
        for b in range(NBUF):
          pltpu.async_copy(x_hbm.at[isv.at[b]], rows[b], sems[b])

      def inner(it, carry):
        jbase = it * NBUF
        for b in range(NBUF):
          jj = jbase + b
          pltpu.make_async_copy(x_hbm.at[isv.at[jj]], rows[b],
                                sems[b]).wait()
          pltpu.sync_copy(rows[b], acc.at[idv.at[jj]], add=True)

          @pl.when(jj + NBUF < iblk)
          def _():
            pltpu.async_copy(x_hbm.at[isv.at[jj + NBUF]], rows[b], sems[b])
        return carry

      lax.fori_loop(0, iblk // NBUF, inner, 0, unroll=False)

      if blk + 1 < NBLK:
        nsv, ndv, nsem = idx[(blk + 1) % 2]
        pltpu.make_async_copy(src_hbm.at[wid, blk + 1], nsv, nsem).wait()
        pltpu.make_async_copy(dst_hbm.at[wid, blk + 1], ndv, nsem).wait()

    plsc.subcore_barrier()
    out0 = pl.multiple_of(c * NP + s * ROWS_PER_S, 8)
    pltpu.sync_copy(acc.at[pl.ds(r0, ROWS_PER_S)],
                    out_hbm.at[pl.ds(out0, ROWS_PER_S)])

  return seg_sum


_seg_sum_cache = {}


def _seg_sum(D):
  if D not in _seg_sum_cache:
    _seg_sum_cache[D] = _make_seg_sum(D, 40 if D > 64 else 80)
  return _seg_sum_cache[D]


NH = N // 2
NPH = NP // 2


def _bd(w_ref):
  w = w_ref[...]
  z = jnp.zeros_like(w)
  return jnp.concatenate(
      [jnp.concatenate([w, z], axis=1), jnp.concatenate([z, w], axis=1)],
      axis=0)


def _t2(v):
  return jnp.concatenate([v, v], axis=1)


def _pack128(y):
  v = jnp.reshape(y, (NH, 2, DIM))
  return jnp.concatenate([v[:, 0, :], v[:, 1, :]], axis=1)


def _unpack128(y128):
  s = jnp.stack([y128[:, :DIM], y128[:, DIM:]], axis=1)
  return jnp.reshape(s, (N, DIM))


def _mlp_bn128(x128, agg_ref, w1_ref, b1_ref, w2_ref, b2_ref,
               gam_ref, bet_ref):
  h = x128 + (agg_ref[pl.ds(0, NH), :] + agg_ref[pl.ds(NPH, NH), :])
  h = jnp.maximum(
      jnp.dot(h, _bd(w1_ref),
              preferred_element_type=jnp.float32) + _t2(b1_ref[...]), 0.0)
  h = jnp.dot(h, _bd(w2_ref),
              preferred_element_type=jnp.float32) + _t2(b2_ref[...])
  h = jnp.maximum(h, 0.0)
  m128 = jnp.mean(h, axis=0, keepdims=True)
  m = _t2(0.5 * (m128[:, :DIM] + m128[:, DIM:]))
  hc = h - m
  v128 = jnp.mean(hc * hc, axis=0, keepdims=True)
  v = 0.5 * (v128[:, :DIM] + v128[:, DIM:]) + BN_EPS
  inv = lax.rsqrt(v)
  inv = inv * (1.5 - 0.5 * v * inv * inv)
  inv = inv * (1.5 - 0.5 * v * inv * inv)
  y = hc * _t2(inv * gam_ref[...]) + _t2(bet_ref[...])
  return y


def _pool128(y, ge_ref, go_ref):
  gids = lax.broadcasted_iota(jnp.int32, (NG, 1), 0)
  me = (ge_ref[...] == gids).astype(jnp.float32)
  mo = (go_ref[...] == gids).astype(jnp.float32)
  pe = jnp.dot(me, y, preferred_element_type=jnp.float32)
  po = jnp.dot(mo, y, preferred_element_type=jnp.float32)
  return pe[:, :DIM] + po[:, DIM:]


def _bn_mlp_pool(x_ref, agg_ref, gi_ref, w1_ref, b1_ref, w2_ref, b2_ref,
                 gam_ref, bet_ref):
  h = x_ref[...] + (agg_ref[pl.ds(0, N), :] + agg_ref[pl.ds(NP, N), :])
  h = jnp.maximum(
      jnp.dot(h, w1_ref[...],
              preferred_element_type=jnp.float32) + b1_ref[...], 0.0)
  h = jnp.dot(h, w2_ref[...],
              preferred_element_type=jnp.float32) + b2_ref[...]
  h = jnp.maximum(h, 0.0)
  mean = jnp.mean(h, axis=0, keepdims=True)
  var = jnp.mean((h - mean) ** 2, axis=0, keepdims=True)
  v = var + BN_EPS
  inv = lax.rsqrt(v)
  inv = inv * (1.5 - 0.5 * v * inv * inv)
  inv = inv * (1.5 - 0.5 * v * inv * inv)
  y = (h - mean) * (inv * gam_ref[...]) + bet_ref[...]
  gids = lax.broadcasted_iota(jnp.int32, (NG, 1), 0)
  mask = (gi_ref[...] == gids).astype(jnp.float32)
  pool = jnp.dot(mask, y, preferred_element_type=jnp.float32)
  return y, pool


def _tc_first_body(x_ref, agg_ref, gi_ref, w1_ref, b1_ref, w2_ref, b2_ref,
                   gam_ref, bet_ref, y_ref, y128_ref, pool_ref):
  y, pool = _bn_mlp_pool(x_ref, agg_ref, gi_ref, w1_ref, b1_ref, w2_ref,
                         b2_ref, gam_ref, bet_ref)
  y_ref[...] = y
  y128_ref[...] = _pack128(y)
  pool_ref[...] = pool


def _tc_mid_body(x_ref, agg_ref, ge_ref, go_ref, w1_ref, b1_ref, w2_ref,
                 b2_ref, gam_ref, bet_ref, y_ref, y128_ref, pool_ref):
  y = _mlp_bn128(x_ref[...], agg_ref, w1_ref, b1_ref, w2_ref, b2_ref,
                 gam_ref, bet_ref)
  y128_ref[...] = y
  y_ref[...] = _unpack128(y)
  pool_ref[...] = _pool128(y, ge_ref, go_ref)


def _tc_last_body(x_ref, agg_ref, ge_ref, go_ref, w1_ref, b1_ref, w2_ref,
                  b2_ref, gam_ref, bet_ref, y0, y1, y2, y3, p0, p1, p2, p3,
                  xs_ref, xo_ref):
  y = _mlp_bn128(x_ref[...], agg_ref, w1_ref, b1_ref, w2_ref, b2_ref,
                 gam_ref, bet_ref)
  pool = _pool128(y, ge_ref, go_ref)
  xs_ref[...] = jnp.concatenate(
      [y0[...], y1[...], y2[...], y3[...], _unpack128(y)], axis=1)
  xo_ref[...] = jnp.concatenate(
      [p0[...], p1[...], p2[...], p3[...], pool], axis=1)


def _wparams(p):
  return (p["W1"], p["b1"].reshape(1, DIM), p["W2"], p["b2"].reshape(1, DIM),
          p["gamma"].reshape(1, DIM), p["beta"].reshape(1, DIM))


def _tc_first(x, agg2, gi2d, p):
  return pl.pallas_call(
      _tc_first_body,
      out_shape=[
          jax.ShapeDtypeStruct((N, DIM), jnp.float32),
          jax.ShapeDtypeStruct((NH, 2 * DIM), jnp.float32),
          jax.ShapeDtypeStruct((NG, DIM), jnp.float32),
      ],
  )(x, agg2, gi2d, *_wparams(p))


def _tc_mid(x128, agg128, ge, go, p):
  return pl.pallas_call(
      _tc_mid_body,
      out_shape=[
          jax.ShapeDtypeStruct((N, DIM), jnp.float32),
          jax.ShapeDtypeStruct((NH, 2 * DIM), jnp.float32),
          jax.ShapeDtypeStruct((NG, DIM), jnp.float32),
      ],
  )(x128, agg128, ge, go, *_wparams(p))


def _tc_last(x128, agg128, ge, go, p, ys, pools):
  return pl.pallas_call(
      _tc_last_body,
      out_shape=[
          jax.ShapeDtypeStruct((N, 5 * DIM), jnp.float32),
          jax.ShapeDtypeStruct((NG, 5 * DIM), jnp.float32),
      ],
  )(x128, agg128, ge, go, *_wparams(p), *ys, *pools)


def kernel(node_features, edge_index, graph_index, params):
  gi2d = graph_index.reshape(1, N)
  ge = graph_index[0::2].reshape(1, NH)
  go = graph_index[1::2].reshape(1, NH)
  d0 = node_features.shape[1]
  zeros = {d: jnp.zeros((NP, d), jnp.float32) for d in (d0, DIM)}
  edge4 = {}
  for d in (d0, DIM):
    chunk = 40 if d > 64 else 80
    nchunks = EDGES_PER_W // chunk
    edge4[d] = (edge_index[0].reshape(NW, NBLK, nchunks // NBLK, chunk),
                edge_index[1].reshape(NW, NBLK, nchunks // NBLK, chunk))

  src4, dst4 = edge4[d0]
  agg2 = _seg_sum(d0)(node_features, src4, dst4, zeros[d0])
  y0, x128, pool0 = _tc_first(node_features, agg2, gi2d, params["layer_0"])
  ys = [y0]
  pools = [pool0]
  src4, dst4 = edge4[DIM]
  for i in (1, 2, 3):
    agg2 = _seg_sum(DIM)(x128.reshape(N, DIM), src4, dst4, zeros[DIM])
    agg128 = agg2.reshape(NC * NPH, 2 * DIM)
    y64, x128, pooled = _tc_mid(x128, agg128, ge, go, params[f"layer_{i}"])
    ys.append(y64)
    pools.append(pooled)
  agg2 = _seg_sum(DIM)(x128.reshape(N, DIM), src4, dst4, zeros[DIM])
  agg128 = agg2.reshape(NC * NPH, 2 * DIM)
  xs_out, x_out = _tc_last(x128, agg128, ge, go, params["layer_4"], ys, pools)
  return x_out, xs_out
